# baseline (speedup 1.0000x reference)
import functools

import jax
import jax.numpy as jnp
from jax import lax
from jax.experimental import pallas as pl
from jax.experimental.pallas import tpu as pltpu

Q = 16
DIM = 64
PACK = 4
KTOP = 10
NSEL = 16
NCAND = 48
BUF = 128
EPS = 1.0


def _fold_slots(x):
    out = x[:, 0:Q]
    for s in range(1, PACK):
        out = jnp.minimum(out, x[:, s * Q:(s + 1) * Q])
    return out


def _knn_kernel(a1_ref, a2_ref, k_ref, dout_ref, iout_ref, dscr_ref,
                *, block_k):
    t = pl.program_id(0)
    rows = block_k // PACK

    @pl.when(t == 0)
    def _init():
        dout_ref[...] = jnp.full((BUF, Q), jnp.inf, jnp.float32)
        iout_ref[...] = jnp.zeros((BUF, Q), jnp.int32)

    kb = jnp.concatenate(
        [k_ref[s * rows:(s + 1) * rows, :] for s in range(PACK)],
        axis=1)
    a1 = a1_ref[...]
    a2 = a2_ref[...]

    kbb = kb.astype(jnp.bfloat16)
    ksqb = (kb * kb).astype(jnp.bfloat16)
    qk = lax.dot_general(kbb, a1, (((1,), (0,)), ((), ())),
                         preferred_element_type=jnp.float32)
    c2 = lax.dot_general(ksqb, a2, (((1,), (0,)), ((), ())),
                         preferred_element_type=jnp.float32)
    d = c2 + qk

    rowi = lax.broadcasted_iota(jnp.int32, (rows, PACK * Q), 0)
    slot = lax.broadcasted_iota(jnp.int32, (rows, PACK * Q), 1) // Q
    base = (t * block_k).astype(jnp.int32)
    gidx = slot * rows + rowi + base
    bufi = lax.broadcasted_iota(jnp.int32, (BUF, Q), 0)
    BIGI = jnp.int32(2**31 - 1)

    tau = dout_ref[KTOP - 1:KTOP, :]
    tau4 = jnp.concatenate([tau] * PACK, axis=1)
    hit = jnp.any(d < tau4 + EPS)

    @pl.when(hit)
    def _merge():
        dscr_ref[...] = d

        def cond(c):
            return (c[0] < NSEL) & c[1]

        def body(c):
            r, _, m_prev, g_prev = c
            dd = dscr_ref[...]
            m_prev4 = jnp.concatenate([m_prev] * PACK, axis=1)
            g_prev4 = jnp.concatenate([g_prev] * PACK, axis=1)
            live = jnp.where((dd > m_prev4) |
                             ((dd == m_prev4) & (gidx > g_prev4)),
                             dd, jnp.inf)
            mcol = jnp.min(live, axis=0, keepdims=True)
            mq = _fold_slots(mcol)
            mq4 = jnp.concatenate([mq] * PACK, axis=1)
            g = jnp.min(jnp.where(live == mq4, gidx, BIGI),
                        axis=0, keepdims=True)
            gq = _fold_slots(g)

            vals = dout_ref[...]
            idxs = iout_ref[...]
            do_q = mq < vals[KTOP - 1:KTOP, :] + EPS
            pos = jnp.sum((vals <= mq).astype(jnp.int32),
                          axis=0, keepdims=True)
            vshift = jnp.concatenate([vals[:1], vals[:-1]], axis=0)
            ishift = jnp.concatenate([idxs[:1], idxs[:-1]], axis=0)
            newv = jnp.where(bufi < pos, vals,
                             jnp.where(bufi == pos, mq, vshift))
            newi = jnp.where(bufi < pos, idxs,
                             jnp.where(bufi == pos, gq, ishift))
            dout_ref[...] = jnp.where(do_q, newv, vals)
            iout_ref[...] = jnp.where(do_q, newi, idxs)
            return r + jnp.int32(1), jnp.any(do_q), mq, gq

        lax.while_loop(cond, body,
                       (jnp.int32(0), True,
                        jnp.full((1, Q), -jnp.inf, jnp.float32),
                        jnp.full((1, Q), -1, jnp.int32)))


def kernel(queries, keys, k):
    nkeys = keys.shape[0]
    block_k = 20000
    assert nkeys % block_k == 0
    nb = nkeys // block_k
    rows = block_k // PACK

    eye = jnp.eye(PACK, dtype=jnp.float32)
    a1 = jnp.einsum("st,dq->sdtq", eye, -2.0 * queries.T).reshape(
        PACK * DIM, PACK * Q).astype(jnp.bfloat16)
    a2 = jnp.einsum("st,dq->sdtq", eye,
                    jnp.ones((DIM, Q), jnp.float32)).reshape(
        PACK * DIM, PACK * Q).astype(jnp.bfloat16)

    _, ipad = pl.pallas_call(
        functools.partial(_knn_kernel, block_k=block_k),
        grid=(nb,),
        in_specs=[
            pl.BlockSpec((PACK * DIM, PACK * Q), lambda t: (0, 0)),
            pl.BlockSpec((PACK * DIM, PACK * Q), lambda t: (0, 0)),
            pl.BlockSpec((block_k, DIM), lambda t: (t, 0)),
        ],
        out_specs=[
            pl.BlockSpec((BUF, Q), lambda t: (0, 0)),
            pl.BlockSpec((BUF, Q), lambda t: (0, 0)),
        ],
        out_shape=[
            jax.ShapeDtypeStruct((BUF, Q), jnp.float32),
            jax.ShapeDtypeStruct((BUF, Q), jnp.int32),
        ],
        scratch_shapes=[pltpu.VMEM((rows, PACK * Q), jnp.float32)],
    )(a1, a2, keys)

    cand = jnp.sort(ipad[:NCAND, :].reshape(-1))
    dup = jnp.concatenate(
        [jnp.zeros((1,), jnp.bool_), cand[1:] == cand[:-1]])
    sub = keys[cand]
    q2 = jnp.sum(queries * queries, axis=1, keepdims=True)
    c2 = jnp.sum(sub * sub, axis=1)[None, :]
    d2 = q2 + c2 - 2.0 * (queries @ sub.T)
    d2 = jnp.where(dup[None, :], jnp.inf, d2)
    neg_vals, pos = lax.top_k(-d2, KTOP)
    D = -neg_vals
    I = cand[pos]
    kth = D[-1, -1]
    return D, I, kth


# speedup vs baseline: 1.5270x; 1.5270x over previous
import functools

import jax
import jax.numpy as jnp
from jax import lax
from jax.experimental import pallas as pl
from jax.experimental.pallas import tpu as pltpu

Q = 16
DIM = 64
PACK = 4
KTOP = 10
NSEL = 24
NCAND = 64
BUF = 128
EPS = 1.5
IDXB = 15
IDXM = (1 << IDXB) - 1


def _fold_slots(x):
    out = x[:, 0:Q]
    for s in range(1, PACK):
        out = jnp.minimum(out, x[:, s * Q:(s + 1) * Q])
    return out


def _knn_kernel(a1_ref, a2_ref, k_ref, dout_ref, iout_ref, iscr_ref,
                *, block_k):
    t = pl.program_id(0)
    rows = block_k // PACK

    @pl.when(t == 0)
    def _init():
        dout_ref[...] = jnp.full((BUF, Q), jnp.inf, jnp.float32)
        iout_ref[...] = jnp.zeros((BUF, Q), jnp.int32)

    kb = jnp.concatenate(
        [k_ref[s * rows:(s + 1) * rows, :] for s in range(PACK)],
        axis=1)
    a1 = a1_ref[...]
    a2 = a2_ref[...]

    kbb = kb.astype(jnp.bfloat16)
    ksqb = (kb * kb).astype(jnp.bfloat16)
    qk = lax.dot_general(kbb, a1, (((1,), (0,)), ((), ())),
                         preferred_element_type=jnp.float32)
    c2 = lax.dot_general(ksqb, a2, (((1,), (0,)), ((), ())),
                         preferred_element_type=jnp.float32)
    d = c2 + qk

    rowi = lax.broadcasted_iota(jnp.int32, (rows, PACK * Q), 0)
    slot = lax.broadcasted_iota(jnp.int32, (rows, PACK * Q), 1) // Q
    lidx = slot * rows + rowi
    s32 = lax.bitcast_convert_type(d, jnp.int32)
    key = s32 ^ (lax.shift_right_arithmetic(s32, 31) &
                 jnp.int32(0x7FFFFFFF))
    comb = (key & jnp.int32(~IDXM)) | lidx
    iscr_ref[...] = comb

    base = (t * block_k).astype(jnp.int32)
    bufi = lax.broadcasted_iota(jnp.int32, (BUF, Q), 0)
    MAXI = jnp.int32(2**31 - 1)

    def cond(c):
        return (c[0] < NSEL) & c[1]

    def body(c):
        r, _, prev = c
        cc = iscr_ref[...]
        prev4 = jnp.concatenate([prev] * PACK, axis=1)
        live = jnp.where(cc > prev4, cc, MAXI)
        mcol = jnp.min(live, axis=0, keepdims=True)
        mc = _fold_slots(mcol)
        kbits = mc & jnp.int32(~IDXM)
        dec = lax.bitcast_convert_type(
            kbits ^ (lax.shift_right_arithmetic(kbits, 31) &
                     jnp.int32(0x7FFFFFFF)), jnp.float32)
        gq = (mc & IDXM) + base

        vals = dout_ref[...]
        idxs = iout_ref[...]
        do_q = dec < vals[KTOP - 1:KTOP, :] + EPS
        pos = jnp.sum((vals <= dec).astype(jnp.int32),
                      axis=0, keepdims=True)
        vshift = jnp.concatenate([vals[:1], vals[:-1]], axis=0)
        ishift = jnp.concatenate([idxs[:1], idxs[:-1]], axis=0)
        newv = jnp.where(bufi < pos, vals,
                         jnp.where(bufi == pos, dec, vshift))
        newi = jnp.where(bufi < pos, idxs,
                         jnp.where(bufi == pos, gq, ishift))
        dout_ref[...] = jnp.where(do_q, newv, vals)
        iout_ref[...] = jnp.where(do_q, newi, idxs)
        return r + jnp.int32(1), jnp.any(do_q), mc

    lax.while_loop(cond, body,
                   (jnp.int32(0), True,
                    jnp.full((1, Q), -(2**31 - 1) - 1, jnp.int32)))


def kernel(queries, keys, k):
    nkeys = keys.shape[0]
    block_k = 20000
    assert nkeys % block_k == 0
    assert block_k <= (1 << IDXB)
    nb = nkeys // block_k
    rows = block_k // PACK

    eye = jnp.eye(PACK, dtype=jnp.float32)
    a1 = jnp.einsum("st,dq->sdtq", eye, -2.0 * queries.T).reshape(
        PACK * DIM, PACK * Q).astype(jnp.bfloat16)
    a2 = jnp.einsum("st,dq->sdtq", eye,
                    jnp.ones((DIM, Q), jnp.float32)).reshape(
        PACK * DIM, PACK * Q).astype(jnp.bfloat16)

    _, ipad = pl.pallas_call(
        functools.partial(_knn_kernel, block_k=block_k),
        grid=(nb,),
        in_specs=[
            pl.BlockSpec((PACK * DIM, PACK * Q), lambda t: (0, 0)),
            pl.BlockSpec((PACK * DIM, PACK * Q), lambda t: (0, 0)),
            pl.BlockSpec((block_k, DIM), lambda t: (t, 0)),
        ],
        out_specs=[
            pl.BlockSpec((BUF, Q), lambda t: (0, 0)),
            pl.BlockSpec((BUF, Q), lambda t: (0, 0)),
        ],
        out_shape=[
            jax.ShapeDtypeStruct((BUF, Q), jnp.float32),
            jax.ShapeDtypeStruct((BUF, Q), jnp.int32),
        ],
        scratch_shapes=[pltpu.VMEM((rows, PACK * Q), jnp.int32)],
    )(a1, a2, keys)

    cand = jnp.sort(ipad[:NCAND, :].reshape(-1))
    dup = jnp.concatenate(
        [jnp.zeros((1,), jnp.bool_), cand[1:] == cand[:-1]])
    sub = keys[cand]
    q2 = jnp.sum(queries * queries, axis=1, keepdims=True)
    c2 = jnp.sum(sub * sub, axis=1)[None, :]
    d2 = q2 + c2 - 2.0 * (queries @ sub.T)
    d2 = jnp.where(dup[None, :], jnp.inf, d2)
    neg_vals, pos = lax.top_k(-d2, KTOP)
    D = -neg_vals
    I = cand[pos]
    kth = D[-1, -1]
    return D, I, kth
